# baseline (speedup 1.0000x reference)
import functools

import jax
import jax.numpy as jnp
from jax import lax
from jax.experimental import pallas as pl
from jax.experimental.pallas import tpu as pltpu
from jax.experimental.pallas import tpu_sc as plsc

_D = 64
_B = 4096 * 50
_NC = 2
_NS = 16
_NW = _NC * _NS
_B_PER_W = _B // _NW
_CHUNK = 800
_NCHUNK = _B_PER_W // _CHUNK

_mesh = plsc.VectorSubcoreMesh(core_axis_name="c", subcore_axis_name="s")


@functools.partial(
    pl.kernel,
    out_type=jax.ShapeDtypeStruct((_B, _D), jnp.float32),
    mesh=_mesh,
    scratch_types=[
        pltpu.VMEM((_CHUNK,), jnp.int32),
        pltpu.VMEM((_CHUNK, _D), jnp.float32),
        pltpu.SemaphoreType.DMA,
    ],
    compiler_params=pltpu.CompilerParams(use_tc_tiling_on_sc=False),
)
def _gather(seq_hbm, table_hbm, out_hbm, idx_v, rows_v, sem):
    wid = lax.axis_index("s") * _NC + lax.axis_index("c")

    def body(i, carry):
        base = wid * _B_PER_W + i * _CHUNK
        pltpu.sync_copy(seq_hbm.at[pl.ds(base, _CHUNK)], idx_v)
        pltpu.async_copy(table_hbm.at[idx_v], rows_v, sem).wait()
        pltpu.sync_copy(rows_v, out_hbm.at[pl.ds(base, _CHUNK)])
        return carry

    lax.fori_loop(0, _NCHUNK, body, 0)


def kernel(seq, W):
    flat = seq.reshape(-1).astype(jnp.int32)
    out = _gather(flat, W)
    return out.reshape(seq.shape + (W.shape[-1],))


# speedup vs baseline: 1.0063x; 1.0063x over previous
import functools

import jax
import jax.numpy as jnp
from jax import lax
from jax.experimental import pallas as pl
from jax.experimental.pallas import tpu as pltpu
from jax.experimental.pallas import tpu_sc as plsc

_D = 64
_B = 4096 * 50
_NC = 2
_NS = 16
_NW = _NC * _NS
_B_PER_W = _B // _NW
_CHUNK = 800
_NCHUNK = _B_PER_W // _CHUNK

_mesh = plsc.VectorSubcoreMesh(core_axis_name="c", subcore_axis_name="s")


@functools.partial(
    pl.kernel,
    out_type=jax.ShapeDtypeStruct((_B, _D), jnp.float32),
    mesh=_mesh,
    scratch_types=[
        pltpu.VMEM((_B_PER_W,), jnp.int32),
        pltpu.VMEM((_CHUNK, _D), jnp.float32),
        pltpu.VMEM((_CHUNK, _D), jnp.float32),
        pltpu.SemaphoreType.DMA,
        pltpu.SemaphoreType.DMA,
        pltpu.SemaphoreType.DMA,
        pltpu.SemaphoreType.DMA,
    ],
    compiler_params=pltpu.CompilerParams(use_tc_tiling_on_sc=False),
)
def _gather(seq_hbm, table_hbm, out_hbm, idx_v, rows0, rows1,
            gsem0, gsem1, ssem0, ssem1):
    wid = lax.axis_index("s") * _NC + lax.axis_index("c")
    base_w = wid * _B_PER_W
    rows = [rows0, rows1]
    gsems = [gsem0, gsem1]
    ssems = [ssem0, ssem1]

    pltpu.sync_copy(seq_hbm.at[pl.ds(base_w, _B_PER_W)], idx_v)

    gcopy = [None] * _NCHUNK
    scopy = [None] * _NCHUNK
    gcopy[0] = pltpu.async_copy(
        table_hbm.at[idx_v.at[pl.ds(0, _CHUNK)]], rows[0], gsems[0])
    for i in range(_NCHUNK):
        b = i % 2
        if i + 1 < _NCHUNK:
            nb = (i + 1) % 2
            if i >= 1:
                scopy[i - 1].wait()
            gcopy[i + 1] = pltpu.async_copy(
                table_hbm.at[idx_v.at[pl.ds((i + 1) * _CHUNK, _CHUNK)]],
                rows[nb], gsems[nb])
        gcopy[i].wait()
        scopy[i] = pltpu.async_copy(
            rows[b], out_hbm.at[pl.ds(base_w + i * _CHUNK, _CHUNK)], ssems[b])
    scopy[_NCHUNK - 2].wait()
    scopy[_NCHUNK - 1].wait()


def kernel(seq, W):
    flat = seq.reshape(-1).astype(jnp.int32)
    out = _gather(flat, W)
    return out.reshape(seq.shape + (W.shape[-1],))


# speedup vs baseline: 1.0141x; 1.0078x over previous
import functools

import jax
import jax.numpy as jnp
from jax import lax
from jax.experimental import pallas as pl
from jax.experimental.pallas import tpu as pltpu
from jax.experimental.pallas import tpu_sc as plsc

_D = 64
_DP = 128
_B = 4096 * 50
_NC = 2
_NS = 16
_NW = _NC * _NS
_B_PER_W = _B // _NW
_CHUNK = 400
_NCHUNK = _B_PER_W // _CHUNK

_mesh = plsc.VectorSubcoreMesh(core_axis_name="c", subcore_axis_name="s")


@functools.partial(
    pl.kernel,
    out_type=jax.ShapeDtypeStruct((_B, _DP), jnp.float32),
    mesh=_mesh,
    scratch_types=[
        pltpu.VMEM((_B_PER_W,), jnp.int32),
        pltpu.VMEM((_CHUNK, _DP), jnp.float32),
        pltpu.VMEM((_CHUNK, _DP), jnp.float32),
        pltpu.SemaphoreType.DMA,
        pltpu.SemaphoreType.DMA,
        pltpu.SemaphoreType.DMA,
        pltpu.SemaphoreType.DMA,
    ],
)
def _gather(seq_hbm, table_hbm, out_hbm, idx_v, rows0, rows1,
            gsem0, gsem1, ssem0, ssem1):
    wid = lax.axis_index("s") * _NC + lax.axis_index("c")
    base_w = wid * _B_PER_W
    rows = [rows0, rows1]
    gsems = [gsem0, gsem1]
    ssems = [ssem0, ssem1]

    pltpu.sync_copy(seq_hbm.at[pl.ds(base_w, _B_PER_W)], idx_v)

    gcopy = [None] * _NCHUNK
    scopy = [None] * _NCHUNK
    gcopy[0] = pltpu.async_copy(
        table_hbm.at[idx_v.at[pl.ds(0, _CHUNK)]], rows[0], gsems[0])
    for i in range(_NCHUNK):
        b = i % 2
        if i + 1 < _NCHUNK:
            nb = (i + 1) % 2
            if i >= 1:
                scopy[i - 1].wait()
            gcopy[i + 1] = pltpu.async_copy(
                table_hbm.at[idx_v.at[pl.ds((i + 1) * _CHUNK, _CHUNK)]],
                rows[nb], gsems[nb])
        gcopy[i].wait()
        scopy[i] = pltpu.async_copy(
            rows[b], out_hbm.at[pl.ds(base_w + i * _CHUNK, _CHUNK)], ssems[b])
    scopy[_NCHUNK - 2].wait()
    scopy[_NCHUNK - 1].wait()


def kernel(seq, W):
    flat = seq.reshape(-1).astype(jnp.int32)
    wp = jnp.pad(W, ((0, 0), (0, _DP - _D)))
    out = _gather(flat, wp)
    return out[:, :_D].reshape(seq.shape + (_D,))
